# baseline (speedup 1.0000x reference)
import jax
import jax.numpy as jnp
from jax import lax
from jax.experimental import pallas as pl
from jax.experimental.pallas import tpu as pltpu
from jax.experimental.pallas import tpu_sc as plsc


_ROWS = 16384
_COLS = 4096
_NC = 2
_NS = 16
_NW = _NC * _NS
_RPW = _ROWS // _NW
_RB = 4
_NBUF = 4
_NCH = _RPW // _RB


def _sc_body(in_hbm, out_hbm, buf, sin, sout):
    wid = lax.axis_index("s") * _NC + lax.axis_index("c")
    base = wid * _RPW

    def _in(c, b):
        row = base + c * _RB
        return pltpu.make_async_copy(
            in_hbm.at[pl.ds(row, _RB), :], buf.at[b], sin.at[b])

    def _out(c, b):
        row = base + c * _RB
        return pltpu.make_async_copy(
            buf.at[b], out_hbm.at[pl.ds(row, _RB), :], sout.at[b])

    for b in range(_NBUF):
        _in(b, b).start()

    def step(it, carry):
        first = it * _NBUF
        for b in range(_NBUF):
            _in(first + b, b).wait()
            _out(first + b, b).start()
        for b in range(_NBUF):
            nc = first + b + _NBUF

            @pl.when(nc < _NCH)
            def _():
                _out(first + b, b).wait()
                _in(nc, b).start()
        return carry

    lax.fori_loop(0, _NCH // _NBUF, step, 0)
    for b in range(_NBUF):
        _out(_NCH - _NBUF + b, b).wait()


def kernel(tensor):
    flat = tensor.reshape(_ROWS, _COLS)
    k = pl.kernel(
        _sc_body,
        out_type=jax.ShapeDtypeStruct((_ROWS, _COLS), jnp.float32),
        mesh=plsc.VectorSubcoreMesh(core_axis_name="c", subcore_axis_name="s"),
        scratch_types=[
            pltpu.VMEM((_NBUF, _RB, _COLS), jnp.float32),
            pltpu.SemaphoreType.DMA((_NBUF,)),
            pltpu.SemaphoreType.DMA((_NBUF,)),
        ],
    )
    out = k(flat)
    return out.reshape(tensor.shape[0], tensor.shape[1], _COLS)


# speedup vs baseline: 2.3668x; 2.3668x over previous
import jax
import jax.numpy as jnp
from jax.experimental import pallas as pl


_B0 = 4
_B1 = 4096
_COLS = 4096
_BLK = 512


def _body(in_ref, out_ref):
    out_ref[...] = in_ref[...].reshape(1, _BLK, _COLS)


def kernel(tensor):
    out = pl.pallas_call(
        _body,
        grid=(_B0, _B1 // _BLK),
        in_specs=[pl.BlockSpec((1, _BLK, 32, 128), lambda i, j: (i, j, 0, 0))],
        out_specs=pl.BlockSpec((1, _BLK, _COLS), lambda i, j: (i, j, 0)),
        out_shape=jax.ShapeDtypeStruct((_B0, _B1, _COLS), jnp.float32),
    )(tensor)
    return out
